# baseline (speedup 1.0000x reference)
import jax
import jax.numpy as jnp
from jax.experimental import pallas as pl
from jax.experimental.pallas import tpu as pltpu


def _expert_matmul_kernel(blk_ref, x_ref, w_ref, b_ref, o_ref):
    del blk_ref
    half = o_ref.shape[1]
    j = pl.program_id(1)
    x = x_ref[...]
    w = w_ref[0, pl.ds(j * half, half), :]
    acc = jax.lax.dot_general(
        x, w, (((1,), (1,)), ((), ())),
        precision=jax.lax.Precision.DEFAULT,
        preferred_element_type=jnp.float32,
    )
    o_ref[...] = acc + b_ref[0]


def kernel(inp, fwd_expert_count, weight, bias):
    num_expert, d_out, d_in = weight.shape
    tokens = inp.shape[0]
    slab = tokens // num_expert
    half = d_out // 2

    offsets = jnp.concatenate(
        [jnp.zeros(1, dtype=jnp.int32), jnp.cumsum(fwd_expert_count).astype(jnp.int32)]
    )
    blk = offsets[:num_expert] // slab

    bias3 = bias.reshape(num_expert, 1, d_out)

    out = pl.pallas_call(
        _expert_matmul_kernel,
        grid_spec=pltpu.PrefetchScalarGridSpec(
            num_scalar_prefetch=1,
            grid=(num_expert, 2),
            in_specs=[
                pl.BlockSpec((slab, d_in), lambda e, j, blk: (blk[e], 0)),
                pl.BlockSpec((1, d_out, d_in), lambda e, j, blk: (e, 0, 0)),
                pl.BlockSpec((1, 1, half), lambda e, j, blk: (e, 0, j)),
            ],
            out_specs=pl.BlockSpec((slab, half), lambda e, j, blk: (e, j)),
        ),
        out_shape=jax.ShapeDtypeStruct((tokens, d_out), jnp.float32),
        compiler_params=pltpu.CompilerParams(
            dimension_semantics=("parallel", "arbitrary"),
        ),
    )(blk, inp, weight, bias3)
    return out


# speedup vs baseline: 1.3853x; 1.3853x over previous
import jax
import jax.numpy as jnp
from jax.experimental import pallas as pl
from jax.experimental.pallas import tpu as pltpu


def _expert_matmul_kernel(blk_ref, x_ref, w_ref, b_ref, o_ref):
    del blk_ref
    x = x_ref[...]
    w = w_ref[0]
    acc = jax.lax.dot_general(
        x, w, (((1,), (1,)), ((), ())),
        precision=jax.lax.Precision.DEFAULT,
        preferred_element_type=jnp.float32,
    )
    o_ref[...] = acc + b_ref[0]


def kernel(inp, fwd_expert_count, weight, bias):
    num_expert, d_out, d_in = weight.shape
    tokens = inp.shape[0]
    slab = tokens // num_expert

    offsets = jnp.concatenate(
        [jnp.zeros(1, dtype=jnp.int32), jnp.cumsum(fwd_expert_count).astype(jnp.int32)]
    )
    blk = offsets[:num_expert] // slab

    bias3 = bias.reshape(num_expert, 1, d_out)

    out = pl.pallas_call(
        _expert_matmul_kernel,
        grid_spec=pltpu.PrefetchScalarGridSpec(
            num_scalar_prefetch=1,
            grid=(num_expert,),
            in_specs=[
                pl.BlockSpec((slab, d_in), lambda e, blk: (blk[e], 0)),
                pl.BlockSpec((1, d_out, d_in), lambda e, blk: (e, 0, 0)),
                pl.BlockSpec((1, 1, d_out), lambda e, blk: (e, 0, 0)),
            ],
            out_specs=pl.BlockSpec((slab, d_out), lambda e, blk: (e, 0)),
        ),
        out_shape=jax.ShapeDtypeStruct((tokens, d_out), jnp.float32),
        compiler_params=pltpu.CompilerParams(
            dimension_semantics=("parallel",),
        ),
    )(blk, inp, weight, bias3)
    return out
